# Initial kernel scaffold; baseline (speedup 1.0000x reference)
import jax
import jax.numpy as jnp
from jax.experimental import pallas as pl


def kernel(x, edge_index, edge_weight, W1, b1, W2, b2):
    raise NotImplementedError("write your pallas kernel here")


# speedup vs baseline: 9.9467x; 9.9467x over previous
import functools

import jax
import jax.numpy as jnp
from jax import lax
from jax.experimental import pallas as pl
from jax.experimental.pallas import tpu as pltpu
from jax.experimental.pallas import tpu_sc as plsc

_N = 10000
_E = 320000
_H = 64
_LANES = 128
_NSUB = 16
_NCORE = 2
_NT = _NSUB * _NCORE
_RP = 2560
_EP = _RP * _LANES
_RGT = _RP // _NT
_RST = _RP // _NSUB
_SL = 640
_NPAD = _SL * _NSUB


@functools.partial(
    pl.kernel,
    out_type=jax.ShapeDtypeStruct((_NPAD,), jnp.float32),
    mesh=plsc.VectorSubcoreMesh(core_axis_name="c", subcore_axis_name="s"),
    compiler_params=pltpu.CompilerParams(needs_layout_passes=False, use_tc_tiling_on_sc=False),
    scratch_types=[
        pltpu.VMEM((_RST, _LANES), jnp.int32),
        pltpu.VMEM((_RST, _LANES), jnp.float32),
        pltpu.VMEM((_SL,), jnp.float32),
        pltpu.VMEM_SHARED((_NPAD,), jnp.float32),
    ],
)
def _deg(col2d, ew2d, deg_out, colb, ewb, dbuf, deg_sp):
    c = lax.axis_index("c")
    s = lax.axis_index("s")

    def fill_ones(i, _):
        dbuf[pl.ds(i * 16, 16)] = jnp.full((16,), 1.0, jnp.float32)
        return _
    lax.fori_loop(0, _SL // 16, fill_ones, None)
    pltpu.sync_copy(dbuf, deg_sp.at[pl.ds(s * _SL, _SL)])
    plsc.subcore_barrier()

    pltpu.sync_copy(col2d.at[pl.ds(s * _RST, _RST)], colb)
    pltpu.sync_copy(ew2d.at[pl.ds(s * _RST, _RST)], ewb)

    def scat(j, _):
        pltpu.sync_copy(ewb.at[j], deg_sp.at[colb.at[j]], add=True)
        return _
    lax.fori_loop(0, _RST, scat, None)
    plsc.subcore_barrier()

    @pl.when(c == 0)
    def _w():
        pltpu.sync_copy(deg_sp.at[pl.ds(s * _SL, _SL)],
                        deg_out.at[pl.ds(s * _SL, _SL)])


@functools.partial(
    pl.kernel,
    out_type=jax.ShapeDtypeStruct((_NCORE, _NPAD, _H), jnp.float32),
    mesh=plsc.VectorSubcoreMesh(core_axis_name="c", subcore_axis_name="s"),
    compiler_params=pltpu.CompilerParams(needs_layout_passes=False, use_tc_tiling_on_sc=False),
    scratch_types=[
        pltpu.VMEM((_RGT, _LANES), jnp.int32),
        pltpu.VMEM((_RGT, _LANES), jnp.int32),
        pltpu.VMEM((_RGT, _LANES), jnp.float32),
        pltpu.VMEM((_NPAD,), jnp.float32),
        pltpu.VMEM((_LANES, _H), jnp.float32),
        pltpu.SemaphoreType.DMA,
        pltpu.VMEM_SHARED((_NPAD, _H), jnp.float32),
    ],
)
def _msg(y, row2d, col2d, ew2d, dinv, out,
         rowb, colb, ewb, dinv_full, gbuf, gsem, acc_sp):
    c = lax.axis_index("c")
    s = lax.axis_index("s")
    gid = c * _NSUB + s

    def z(i, _):
        for q in range(4):
            gbuf[i, pl.ds(q * 16, 16)] = jnp.zeros((16,), jnp.float32)
        return _
    lax.fori_loop(0, _LANES, z, None)
    for k in range(_SL // _LANES):
        pltpu.sync_copy(gbuf, acc_sp.at[pl.ds(s * _SL + k * _LANES, _LANES)])
    plsc.subcore_barrier()

    pltpu.sync_copy(dinv, dinv_full)
    pltpu.sync_copy(row2d.at[pl.ds(gid * _RGT, _RGT)], rowb)
    pltpu.sync_copy(col2d.at[pl.ds(gid * _RGT, _RGT)], colb)
    pltpu.sync_copy(ew2d.at[pl.ds(gid * _RGT, _RGT)], ewb)

    def row_body(j, _):
        pltpu.async_copy(y.at[rowb.at[j]], gbuf, gsem).wait()

        def g8(g, _2):
            r16 = rowb[j, pl.ds(g * 16, 16)]
            c16 = colb[j, pl.ds(g * 16, 16)]
            w16 = ewb[j, pl.ds(g * 16, 16)]
            a = plsc.load_gather(dinv_full, [r16])
            b = plsc.load_gather(dinv_full, [c16])
            n16 = a * w16 * b
            for e in range(16):
                sc_ = n16[e]
                base = g * 16 + e
                for q in range(4):
                    sl = pl.ds(q * 16, 16)
                    gbuf[base, sl] = gbuf[base, sl] * sc_
            return _2
        lax.fori_loop(0, 8, g8, None)
        pltpu.sync_copy(gbuf, acc_sp.at[colb.at[j]], add=True)
        return _
    lax.fori_loop(0, _RGT, row_body, None)
    plsc.subcore_barrier()
    pltpu.sync_copy(acc_sp.at[pl.ds(s * _SL, _SL)],
                    out.at[c, pl.ds(s * _SL, _SL)])


def _mm_body(x_ref, w_ref, o_ref):
    o_ref[...] = jnp.dot(x_ref[...], w_ref[...],
                         preferred_element_type=jnp.float32)


def _mm(x, w, bm):
    n, d = x.shape
    h = w.shape[1]
    return pl.pallas_call(
        _mm_body,
        grid=(n // bm,),
        in_specs=[pl.BlockSpec((bm, d), lambda i: (i, 0)),
                  pl.BlockSpec((d, h), lambda i: (0, 0))],
        out_specs=pl.BlockSpec((bm, h), lambda i: (i, 0)),
        out_shape=jax.ShapeDtypeStruct((n, h), jnp.float32),
    )(x, w)


def _dinv_body(deg_ref, o_ref):
    o_ref[...] = lax.rsqrt(deg_ref[...])


def _dinv_tc(deg2d):
    return pl.pallas_call(
        _dinv_body,
        out_shape=jax.ShapeDtypeStruct(deg2d.shape, jnp.float32),
    )(deg2d)


def _finish1_body(p0, p1, z1, dinv, b1, w2, o_ref):
    d = dinv[...]
    h = p0[...] + p1[...] + d * d * z1[...] + b1[...]
    h = jnp.maximum(h, 0.0)
    o_ref[...] = jnp.dot(h, w2[...], preferred_element_type=jnp.float32)


def _finish1(p0, p1, z1, dinv, b1, w2, bm):
    n, hdim = z1.shape
    cdim = w2.shape[1]
    return pl.pallas_call(
        _finish1_body,
        grid=(n // bm,),
        in_specs=[pl.BlockSpec((bm, hdim), lambda i: (i, 0)),
                  pl.BlockSpec((bm, hdim), lambda i: (i, 0)),
                  pl.BlockSpec((bm, hdim), lambda i: (i, 0)),
                  pl.BlockSpec((bm, 1), lambda i: (i, 0)),
                  pl.BlockSpec((1, hdim), lambda i: (0, 0)),
                  pl.BlockSpec((hdim, cdim), lambda i: (0, 0))],
        out_specs=pl.BlockSpec((bm, cdim), lambda i: (i, 0)),
        out_shape=jax.ShapeDtypeStruct((n, cdim), jnp.float32),
    )(p0, p1, z1, dinv, b1, w2)


def _finish2_body(p0, p1, z2, dinv, b2, o_ref):
    d = dinv[...]
    t = p0[...] + p1[...] + d * d * z2[...] + b2[...]
    m = jnp.max(t, axis=1, keepdims=True)
    e = jnp.exp(t - m)
    o_ref[...] = t - m - jnp.log(jnp.sum(e, axis=1, keepdims=True))


def _finish2(p0, p1, z2, dinv, b2, bm):
    n, cdim = z2.shape
    return pl.pallas_call(
        _finish2_body,
        grid=(n // bm,),
        in_specs=[pl.BlockSpec((bm, cdim), lambda i: (i, 0)),
                  pl.BlockSpec((bm, cdim), lambda i: (i, 0)),
                  pl.BlockSpec((bm, cdim), lambda i: (i, 0)),
                  pl.BlockSpec((bm, 1), lambda i: (i, 0)),
                  pl.BlockSpec((1, cdim), lambda i: (0, 0))],
        out_specs=pl.BlockSpec((bm, cdim), lambda i: (i, 0)),
        out_shape=jax.ShapeDtypeStruct((n, cdim), jnp.float32),
    )(p0, p1, z2, dinv, b2)


def kernel(x, edge_index, edge_weight, W1, b1, W2, b2):
    pad = _EP - _E
    row2d = jnp.pad(edge_index[0], (0, pad)).reshape(_RP, _LANES)
    col2d = jnp.pad(edge_index[1], (0, pad)).reshape(_RP, _LANES)
    ew2d = jnp.pad(edge_weight, (0, pad)).reshape(_RP, _LANES)

    z1 = _mm(x, W1, 1000)
    deg_pad = _deg(col2d, ew2d)
    dinv_pad = _dinv_tc(deg_pad.reshape(80, 128)).reshape(_NPAD)
    p1 = _msg(z1, row2d, col2d, ew2d, dinv_pad)
    dinv = dinv_pad[:_N].reshape(_N, 1)
    z2 = _finish1(p1[0, :_N], p1[1, :_N], z1, dinv,
                  b1.reshape(1, -1), W2, 1000)
    p2 = _msg(z2, row2d, col2d, ew2d, dinv_pad)
    return _finish2(p2[0, :_N], p2[1, :_N], z2, dinv,
                    b2.reshape(1, -1), 1000)
